# baseline (speedup 1.0000x reference)
import jax
import jax.numpy as jnp
from jax import lax
from jax.experimental import pallas as pl
from jax.experimental.pallas import tpu as pltpu
from jax.experimental.pallas import tpu_sc as plsc

N = 16384
C = 1000

_info = plsc.get_sparse_core_info()
_NC = _info.num_cores
_NS = _info.num_subcores
_L = _info.num_lanes
_NW = _NC * _NS
_STRIPE = N // _NW
_BAND = 16
_NBUF = 4
_CHUNKS = [(i * _BAND, min(_BAND, C - i * _BAND))
           for i in range((C + _BAND - 1) // _BAND)]


def _one_hot_t_body(x_hbm, out_hbm, x_v, b0, b1, b2, b3, s0, s1, s2, s3):
    wid = lax.axis_index("s") * _NC + lax.axis_index("c")
    col0 = wid * _STRIPE

    pltpu.sync_copy(x_hbm.at[pl.ds(col0, _STRIPE)], x_v)

    bufs = (b0, b1, b2, b3)
    sems = (s0, s1, s2, s3)
    zeros = jnp.zeros((_L,), jnp.int32)

    def _zero_row(r, _):
        for b in bufs:
            for k in range(_STRIPE // _L):
                b[r, pl.ds(k * _L, _L)] = zeros
        return 0

    lax.fori_loop(0, _BAND, _zero_row, 0)

    iota = lax.iota(jnp.int32, _L)
    ones = jnp.ones((_L,), jnp.int32)
    copies = [None] * _NBUF

    def _scatter(buf, cls0, ncls, vals):
        def _body(g, _):
            xv = x_v[pl.ds(g * _L, _L)]
            m = (xv >= cls0) & (xv < cls0 + ncls)
            plsc.store_scatter(buf, [xv - cls0, g * _L + iota], vals, mask=m)
            return 0

        lax.fori_loop(0, _STRIPE // _L, _body, 0)

    for c, (cls0, ncls) in enumerate(_CHUNKS):
        b = c % _NBUF
        if c >= _NBUF:
            copies[b].wait()
            pcls0, pncls = _CHUNKS[c - _NBUF]
            _scatter(bufs[b], pcls0, pncls, zeros)
        _scatter(bufs[b], cls0, ncls, ones)
        dst = out_hbm.at[pl.ds(cls0, ncls), pl.ds(col0, _STRIPE)]
        copies[b] = pltpu.async_copy(bufs[b].at[pl.ds(0, ncls), :], dst,
                                     sems[b])

    for cp in copies:
        cp.wait()


_one_hot_t = pl.kernel(
    _one_hot_t_body,
    out_type=jax.ShapeDtypeStruct((C, N), jnp.int32),
    mesh=plsc.VectorSubcoreMesh(core_axis_name="c", subcore_axis_name="s"),
    scratch_types=(
        [pltpu.VMEM((_STRIPE,), jnp.int32)]
        + [pltpu.VMEM((_BAND, _STRIPE), jnp.int32)] * _NBUF
        + [pltpu.SemaphoreType.DMA] * _NBUF
    ),
    compiler_params=pltpu.CompilerParams(
        needs_layout_passes=False, use_tc_tiling_on_sc=True),
)


@jax.jit
def kernel(x):
    return _one_hot_t(x).T


# speedup vs baseline: 1.4672x; 1.4672x over previous
import jax
import jax.numpy as jnp
from jax import lax
from jax.experimental import pallas as pl
from jax.experimental.pallas import tpu as pltpu
from jax.experimental.pallas import tpu_sc as plsc

N = 16384
C = 1000

_info = plsc.get_sparse_core_info()
_NC = _info.num_cores
_NS = _info.num_subcores
_L = _info.num_lanes
_NW = _NC * _NS
_STRIPE = N // _NW
_BAND = 64
_NBUF = 3
_CHUNKS = [(i * _BAND, min(_BAND, C - i * _BAND))
           for i in range((C + _BAND - 1) // _BAND)]


def _one_hot_t_body(x_hbm, out_hbm, x_v, b0, b1, b2, s0, s1, s2):
    wid = lax.axis_index("s") * _NC + lax.axis_index("c")
    col0 = wid * _STRIPE

    pltpu.sync_copy(x_hbm.at[pl.ds(col0, _STRIPE)], x_v)

    bufs = (b0, b1, b2)
    sems = (s0, s1, s2)
    zeros = jnp.zeros((_L,), jnp.int32)

    def _zero_row(r, _):
        for b in bufs:
            for k in range(_STRIPE // _L):
                b[r, pl.ds(k * _L, _L)] = zeros
        return 0

    lax.fori_loop(0, _BAND, _zero_row, 0)

    iota = lax.iota(jnp.int32, _L)
    ones = jnp.ones((_L,), jnp.int32)
    copies = [None] * _NBUF

    def _scatter(buf, cls0, ncls, vals):
        def _body(g, _):
            xv = x_v[pl.ds(g * _L, _L)]
            m = (xv >= cls0) & (xv < cls0 + ncls)
            plsc.store_scatter(buf, [xv - cls0, g * _L + iota], vals, mask=m)
            return 0

        lax.fori_loop(0, _STRIPE // _L, _body, 0)

    for c, (cls0, ncls) in enumerate(_CHUNKS):
        b = c % _NBUF
        if c >= _NBUF:
            copies[b].wait()
            pcls0, pncls = _CHUNKS[c - _NBUF]
            _scatter(bufs[b], pcls0, pncls, zeros)
        _scatter(bufs[b], cls0, ncls, ones)
        dst = out_hbm.at[pl.ds(cls0, ncls), pl.ds(col0, _STRIPE)]
        copies[b] = pltpu.async_copy(bufs[b].at[pl.ds(0, ncls), :], dst,
                                     sems[b])

    for cp in copies:
        cp.wait()


_one_hot_t = pl.kernel(
    _one_hot_t_body,
    out_type=jax.ShapeDtypeStruct((C, N), jnp.int32),
    mesh=plsc.VectorSubcoreMesh(core_axis_name="c", subcore_axis_name="s"),
    scratch_types=(
        [pltpu.VMEM((_STRIPE,), jnp.int32)]
        + [pltpu.VMEM((_BAND, _STRIPE), jnp.int32)] * _NBUF
        + [pltpu.SemaphoreType.DMA] * _NBUF
    ),
    compiler_params=pltpu.CompilerParams(
        needs_layout_passes=False, use_tc_tiling_on_sc=True),
)


@jax.jit
def kernel(x):
    return _one_hot_t(x).T


# speedup vs baseline: 1.5602x; 1.0634x over previous
import jax
import jax.numpy as jnp
from jax import lax
from jax.experimental import pallas as pl
from jax.experimental.pallas import tpu as pltpu
from jax.experimental.pallas import tpu_sc as plsc

N = 16384
C = 1000

_info = plsc.get_sparse_core_info()
_NC = _info.num_cores
_NS = _info.num_subcores
_L = _info.num_lanes
_NW = _NC * _NS
_STRIPE = N // _NW
_BAND = 96
_NBUF = 2
_CHUNKS = [(i * _BAND, min(_BAND, C - i * _BAND))
           for i in range((C + _BAND - 1) // _BAND)]


def _one_hot_t_body(x_hbm, out_hbm, x_v, b0, b1, s0, s1):
    wid = lax.axis_index("s") * _NC + lax.axis_index("c")
    col0 = wid * _STRIPE

    pltpu.sync_copy(x_hbm.at[pl.ds(col0, _STRIPE)], x_v)

    bufs = (b0, b1)
    sems = (s0, s1)
    zeros = jnp.zeros((_L,), jnp.int32)

    def _zero_buf(buf):
        def _zero_row(r, _):
            for k in range(_STRIPE // _L):
                buf[r, pl.ds(k * _L, _L)] = zeros
            return 0

        lax.fori_loop(0, _BAND, _zero_row, 0)

    iota = lax.iota(jnp.int32, _L)
    ones = jnp.ones((_L,), jnp.int32)
    copies = [None] * _NBUF

    def _scatter(buf, cls0, ncls, vals):
        def _body(g, _):
            xv = x_v[pl.ds(g * _L, _L)]
            m = (xv >= cls0) & (xv < cls0 + ncls)
            plsc.store_scatter(buf, [xv - cls0, g * _L + iota], vals, mask=m)
            return 0

        lax.fori_loop(0, _STRIPE // _L, _body, 0)

    for c, (cls0, ncls) in enumerate(_CHUNKS):
        b = c % _NBUF
        if c < _NBUF:
            _zero_buf(bufs[b])
        else:
            copies[b].wait()
            pcls0, pncls = _CHUNKS[c - _NBUF]
            _scatter(bufs[b], pcls0, pncls, zeros)
        _scatter(bufs[b], cls0, ncls, ones)
        dst = out_hbm.at[pl.ds(cls0, ncls), pl.ds(col0, _STRIPE)]
        copies[b] = pltpu.async_copy(bufs[b].at[pl.ds(0, ncls), :], dst,
                                     sems[b])

    for cp in copies:
        cp.wait()


_one_hot_t = pl.kernel(
    _one_hot_t_body,
    out_type=jax.ShapeDtypeStruct((C, N), jnp.int32),
    mesh=plsc.VectorSubcoreMesh(core_axis_name="c", subcore_axis_name="s"),
    scratch_types=(
        [pltpu.VMEM((_STRIPE,), jnp.int32)]
        + [pltpu.VMEM((_BAND, _STRIPE), jnp.int32)] * _NBUF
        + [pltpu.SemaphoreType.DMA] * _NBUF
    ),
    compiler_params=pltpu.CompilerParams(
        needs_layout_passes=False, use_tc_tiling_on_sc=True),
)


@jax.jit
def kernel(x):
    return _one_hot_t(x).T


# speedup vs baseline: 1.5689x; 1.0056x over previous
import jax
import jax.numpy as jnp
from jax import lax
from jax.experimental import pallas as pl
from jax.experimental.pallas import tpu as pltpu
from jax.experimental.pallas import tpu_sc as plsc

N = 16384
C = 1000

_info = plsc.get_sparse_core_info()
_NC = _info.num_cores
_NS = _info.num_subcores
_L = _info.num_lanes
_NW = _NC * _NS
_STRIPE = N // _NW
_BAND = 120
_NBUF = 2
_CHUNKS = [(i * _BAND, min(_BAND, C - i * _BAND))
           for i in range((C + _BAND - 1) // _BAND)]


def _one_hot_t_body(x_hbm, out_hbm, x_v, b0, b1, s0, s1):
    wid = lax.axis_index("s") * _NC + lax.axis_index("c")
    col0 = wid * _STRIPE

    pltpu.sync_copy(x_hbm.at[pl.ds(col0, _STRIPE)], x_v)

    bufs = (b0, b1)
    sems = (s0, s1)
    zeros = jnp.zeros((_L,), jnp.int32)

    def _zero_buf(buf):
        def _zero_row(r, _):
            for k in range(_STRIPE // _L):
                buf[r, pl.ds(k * _L, _L)] = zeros
            return 0

        lax.fori_loop(0, _BAND, _zero_row, 0)

    iota = lax.iota(jnp.int32, _L)
    ones = jnp.ones((_L,), jnp.int32)
    copies = [None] * _NBUF

    def _scatter(buf, cls0, ncls, vals):
        def _body(g, _):
            xv = x_v[pl.ds(g * _L, _L)]
            m = (xv >= cls0) & (xv < cls0 + ncls)
            plsc.store_scatter(buf, [xv - cls0, g * _L + iota], vals, mask=m)
            return 0

        lax.fori_loop(0, _STRIPE // _L, _body, 0)

    for c, (cls0, ncls) in enumerate(_CHUNKS):
        b = c % _NBUF
        if c < _NBUF:
            _zero_buf(bufs[b])
        else:
            copies[b].wait()
            pcls0, pncls = _CHUNKS[c - _NBUF]
            _scatter(bufs[b], pcls0, pncls, zeros)
        _scatter(bufs[b], cls0, ncls, ones)
        dst = out_hbm.at[pl.ds(cls0, ncls), pl.ds(col0, _STRIPE)]
        copies[b] = pltpu.async_copy(bufs[b].at[pl.ds(0, ncls), :], dst,
                                     sems[b])

    for cp in copies:
        cp.wait()


_one_hot_t = pl.kernel(
    _one_hot_t_body,
    out_type=jax.ShapeDtypeStruct((C, N), jnp.int32),
    mesh=plsc.VectorSubcoreMesh(core_axis_name="c", subcore_axis_name="s"),
    scratch_types=(
        [pltpu.VMEM((_STRIPE,), jnp.int32)]
        + [pltpu.VMEM((_BAND, _STRIPE), jnp.int32)] * _NBUF
        + [pltpu.SemaphoreType.DMA] * _NBUF
    ),
    compiler_params=pltpu.CompilerParams(
        needs_layout_passes=False, use_tc_tiling_on_sc=True),
)


@jax.jit
def kernel(x):
    return _one_hot_t(x).T
